# Initial kernel scaffold; baseline (speedup 1.0000x reference)
import jax
import jax.numpy as jnp
from jax.experimental import pallas as pl


def kernel(node_pos, edge_index, apsp, batch_index, edge_pair_index):
    raise NotImplementedError("write your pallas kernel here")


# speedup vs baseline: 85.6039x; 85.6039x over previous
import functools

import jax
import jax.numpy as jnp
from jax import lax
from jax.experimental import pallas as pl
from jax.experimental.pallas import tpu as pltpu
from jax.experimental.pallas import tpu_sc as plsc

_EPS = 1e-7
_NN = 100000
_NP = 6400000
_NG = 128
_NC = 2
_NS = 16
_NW = _NC * _NS
_P = _NP // _NW
_C = 800
_NCH = _P // _C
_BC = 2000
_NBCH = _NN // _BC

_mesh = plsc.VectorSubcoreMesh(core_axis_name="c", subcore_axis_name="s")
_cparams = pltpu.CompilerParams(needs_layout_passes=False)


def _worker_id():
    return lax.axis_index("s") * _NC + lax.axis_index("c")


@functools.partial(
    pl.kernel,
    out_type=(
        jax.ShapeDtypeStruct((_NP,), jnp.float32),
        jax.ShapeDtypeStruct((_NP,), jnp.float32),
        jax.ShapeDtypeStruct((_NP,), jnp.float32),
    ),
    mesh=_mesh,
    compiler_params=_cparams,
    scratch_types=[
        pltpu.VMEM((_NN,), jnp.float32),
        [pltpu.VMEM((_C,), jnp.int32)] * 4,
        [pltpu.VMEM((_C,), jnp.float32)] * 3,
        pltpu.SemaphoreType.DMA,
    ],
)
def _pass1(xcol, s1_h, s2_h, e1_h, e2_h, a_out, b_out, c_out, xtab, idx, abc, sem):
    base = _worker_id() * _P
    pltpu.sync_copy(xcol, xtab)

    def chunk(i, carry):
        st = base + i * _C
        dsi = [
            pltpu.async_copy(src.at[pl.ds(st, _C)], idx[r], sem)
            for r, src in enumerate((s1_h, s2_h, e1_h, e2_h))
        ]
        for d in dsi:
            d.wait()

        def vreg(j, carry2):
            o = j * 16
            s1 = idx[0][pl.ds(o, 16)]
            s2 = idx[1][pl.ds(o, 16)]
            e1 = idx[2][pl.ds(o, 16)]
            e2 = idx[3][pl.ds(o, 16)]
            xs1 = plsc.load_gather(xtab, [s1])
            xs2 = plsc.load_gather(xtab, [s2])
            xe1 = plsc.load_gather(xtab, [e1])
            xe2 = plsc.load_gather(xtab, [e2])
            abc[0][pl.ds(o, 16)] = xe2 - xs2
            abc[1][pl.ds(o, 16)] = xs1 - xs2
            abc[2][pl.ds(o, 16)] = xe1 - xs2
            return carry2

        lax.fori_loop(0, _C // 16, vreg, 0)
        dso = [
            pltpu.async_copy(abc[k], dst.at[pl.ds(st, _C)], sem)
            for k, dst in enumerate((a_out, b_out, c_out))
        ]
        for d in dso:
            d.wait()
        return carry

    lax.fori_loop(0, _NCH, chunk, 0)


@functools.partial(
    pl.kernel,
    out_type=jax.ShapeDtypeStruct((_NW * _NG,), jnp.float32),
    mesh=_mesh,
    compiler_params=_cparams,
    scratch_types=[
        pltpu.VMEM((_NN,), jnp.float32),
        pltpu.VMEM((_BC,), jnp.int32),
        pltpu.VMEM((16, _NG), jnp.int32),
        pltpu.VMEM((_NG,), jnp.int32),
        [pltpu.VMEM((_C,), jnp.int32)] * 4,
        [pltpu.VMEM((_C,), jnp.float32)] * 3,
        pltpu.VMEM((16, _NG), jnp.float32),
        pltpu.VMEM((_NG,), jnp.float32),
        pltpu.SemaphoreType.DMA,
    ],
)
def _pass2(ycol, batch, s1_h, s2_h, e1_h, e2_h, a_in, b_in, c_in, out,
           ytab, bch, cnt, bnd, idx, abc, hist, orow, sem):
    wid = _worker_id()
    base = wid * _P
    pltpu.sync_copy(ycol, ytab)
    lane = lax.iota(jnp.int32, 16)
    onesi = jnp.ones((16,), jnp.int32)
    zi = jnp.zeros((16,), jnp.int32)
    zf = jnp.zeros((16,), jnp.float32)
    for l in range(16):
        for gb in range(8):
            cnt[l, pl.ds(gb * 16, 16)] = zi
            hist[l, pl.ds(gb * 16, 16)] = zf

    def bchunk(i, carry):
        pltpu.sync_copy(batch.at[pl.ds(i * _BC, _BC)], bch)

        def v(j, carry2):
            bv = bch[pl.ds(j * 16, 16)]
            plsc.addupdate_scatter(cnt, [lane, bv], onesi)
            return carry2

        lax.fori_loop(0, _BC // 16, v, 0)
        return carry

    lax.fori_loop(0, _NBCH, bchunk, 0)

    carry = jnp.int32(0)
    for gb in range(8):
        acc = zi
        for l in range(16):
            acc = acc + cnt[l, pl.ds(gb * 16, 16)]
        blk = plsc.cumsum(acc) + carry
        bnd[pl.ds(gb * 16, 16)] = blk
        carry = blk[15]

    def chunk(i, carry3):
        st = base + i * _C
        dsi = [
            pltpu.async_copy(src.at[pl.ds(st, _C)], idx[r], sem)
            for r, src in enumerate((s1_h, s2_h, e1_h, e2_h))
        ]
        dsa = [
            pltpu.async_copy(src.at[pl.ds(st, _C)], abc[k], sem)
            for k, src in enumerate((a_in, b_in, c_in))
        ]
        for d in dsi + dsa:
            d.wait()

        def vreg(j, carry2):
            o = j * 16
            s1 = idx[0][pl.ds(o, 16)]
            s2 = idx[1][pl.ds(o, 16)]
            e1 = idx[2][pl.ds(o, 16)]
            e2 = idx[3][pl.ds(o, 16)]
            ys1 = plsc.load_gather(ytab, [s1])
            ys2 = plsc.load_gather(ytab, [s2])
            ye1 = plsc.load_gather(ytab, [e1])
            ye2 = plsc.load_gather(ytab, [e2])
            ap = ye2 - ys2
            bp = ys1 - ys2
            cp = ye1 - ys2
            av = abc[0][pl.ds(o, 16)]
            bv = abc[1][pl.ds(o, 16)]
            cv = abc[2][pl.ds(o, 16)]
            d1 = av * bp - ap * bv
            d2 = av * cp - ap * cv
            d3 = bv * cp - bp * cv
            d4 = d1 - d2 + d3
            cross = (d1 * d2 < -_EPS) & (d3 * d4 < -_EPS)
            xing = jnp.where(cross, 1.0, 0.0).astype(jnp.float32)
            lo = zi
            for stp in (64, 32, 16, 8, 4, 2, 1):
                mid = lo + stp
                bb = plsc.load_gather(bnd, [mid - 1])
                lo = jnp.where(bb <= s1, mid, lo)
            plsc.addupdate_scatter(hist, [lane, lo], xing)
            return carry2

        lax.fori_loop(0, _C // 16, vreg, 0)
        return carry3

    lax.fori_loop(0, _NCH, chunk, 0)

    for gb in range(8):
        accf = zf
        for l in range(16):
            accf = accf + hist[l, pl.ds(gb * 16, 16)]
        orow[pl.ds(gb * 16, 16)] = accf
    pltpu.sync_copy(orow, out.at[pl.ds(wid * _NG, _NG)])


def kernel(node_pos, edge_index, apsp, batch_index, edge_pair_index):
    del edge_index, apsp
    xcol = node_pos[:, 0]
    ycol = node_pos[:, 1]
    epi = edge_pair_index.reshape(4, _NP)
    s1, s2, e1, e2 = epi[0], epi[1], epi[2], epi[3]
    a, b, c = _pass1(xcol, s1, s2, e1, e2)
    parts = _pass2(ycol, batch_index, s1, s2, e1, e2, a, b, c)
    return parts.reshape(_NW, _NG).sum(axis=0)


# speedup vs baseline: 117.3373x; 1.3707x over previous
import functools

import jax
import jax.numpy as jnp
from jax import lax
from jax.experimental import pallas as pl
from jax.experimental.pallas import tpu as pltpu
from jax.experimental.pallas import tpu_sc as plsc

_EPS = 1e-7
_NN = 100000
_NP = 6400000
_NG = 128
_NC = 2
_NS = 16
_NW = _NC * _NS
_P = _NP // _NW
_C = 400
_NCH = _P // _C
_U = 5
_BC = 2000
_NBCH = _NN // _BC

_mesh = plsc.VectorSubcoreMesh(core_axis_name="c", subcore_axis_name="s")
_cparams = pltpu.CompilerParams(needs_layout_passes=False)


def _worker_id():
    return lax.axis_index("s") * _NC + lax.axis_index("c")


@functools.partial(
    pl.kernel,
    out_type=(
        jax.ShapeDtypeStruct((_NP,), jnp.float32),
        jax.ShapeDtypeStruct((_NP,), jnp.float32),
        jax.ShapeDtypeStruct((_NP,), jnp.float32),
    ),
    mesh=_mesh,
    compiler_params=_cparams,
    scratch_types=[
        pltpu.VMEM((_NN,), jnp.float32),
        [pltpu.VMEM((_C,), jnp.int32)] * 8,
        [pltpu.VMEM((_C,), jnp.float32)] * 6,
        [pltpu.SemaphoreType.DMA] * 4,
    ],
)
def _pass1(epi, xcol, a_out, b_out, c_out, xtab, idx, abc, sems):
    base = _worker_id() * _P
    pltpu.sync_copy(xcol, xtab)
    semi = sems[:2]
    semo = sems[2:]
    outs = (a_out, b_out, c_out)

    def issue_in(s, st):
        for r in range(4):
            pltpu.async_copy(epi.at[pl.ds(r * _NP + st, _C)],
                             idx[s * 4 + r], semi[s])

    def wait_in(s):
        for r in range(4):
            pltpu.make_async_copy(epi.at[pl.ds(0, _C)],
                                  idx[s * 4 + r], semi[s]).wait()

    def issue_out(s, st):
        for k in range(3):
            pltpu.async_copy(abc[s * 3 + k], outs[k].at[pl.ds(st, _C)],
                             semo[s])

    def wait_out(s):
        for k in range(3):
            pltpu.make_async_copy(abc[s * 3 + k], outs[k].at[pl.ds(0, _C)],
                                  semo[s]).wait()

    def compute(s):
        def body(j, carry):
            o = j * (16 * _U)
            for u in range(_U):
                oo = o + u * 16
                s1 = idx[s * 4 + 0][pl.ds(oo, 16)]
                s2 = idx[s * 4 + 1][pl.ds(oo, 16)]
                e1 = idx[s * 4 + 2][pl.ds(oo, 16)]
                e2 = idx[s * 4 + 3][pl.ds(oo, 16)]
                xs1 = plsc.load_gather(xtab, [s1])
                xs2 = plsc.load_gather(xtab, [s2])
                xe1 = plsc.load_gather(xtab, [e1])
                xe2 = plsc.load_gather(xtab, [e2])
                abc[s * 3 + 0][pl.ds(oo, 16)] = xe2 - xs2
                abc[s * 3 + 1][pl.ds(oo, 16)] = xs1 - xs2
                abc[s * 3 + 2][pl.ds(oo, 16)] = xe1 - xs2
            return carry

        lax.fori_loop(0, _C // (16 * _U), body, 0)

    issue_in(0, base)
    issue_in(1, base + _C)

    def h_iter(h, carry):
        st0 = base + (2 * h) * _C
        wait_in(0)

        @pl.when(h > 0)
        def _wo0():
            wait_out(0)

        compute(0)
        issue_out(0, st0)
        issue_in(0, st0 + 2 * _C)
        st1 = st0 + _C
        wait_in(1)

        @pl.when(h > 0)
        def _wo1():
            wait_out(1)

        compute(1)
        issue_out(1, st1)
        issue_in(1, st1 + 2 * _C)
        return carry

    lax.fori_loop(0, _NCH // 2 - 1, h_iter, 0)

    st0 = base + (_NCH - 2) * _C
    wait_in(0)
    wait_out(0)
    compute(0)
    issue_out(0, st0)
    wait_in(1)
    wait_out(1)
    compute(1)
    issue_out(1, st0 + _C)
    wait_out(0)
    wait_out(1)


@functools.partial(
    pl.kernel,
    out_type=jax.ShapeDtypeStruct((_NW * _NG,), jnp.float32),
    mesh=_mesh,
    compiler_params=_cparams,
    scratch_types=[
        pltpu.VMEM((_NN,), jnp.float32),
        pltpu.VMEM((_BC,), jnp.int32),
        pltpu.VMEM((16, _NG), jnp.int32),
        pltpu.VMEM((_NG,), jnp.int32),
        [pltpu.VMEM((_C,), jnp.int32)] * 8,
        [pltpu.VMEM((_C,), jnp.float32)] * 6,
        pltpu.VMEM((16, _NG), jnp.float32),
        pltpu.VMEM((_NG,), jnp.float32),
        [pltpu.SemaphoreType.DMA] * 2,
    ],
)
def _pass2(epi, ycol, batch, a_in, b_in, c_in, out,
           ytab, bch, cnt, bnd, idx, abc, hist, orow, semi):
    wid = _worker_id()
    base = wid * _P
    pltpu.sync_copy(ycol, ytab)
    lane = lax.iota(jnp.int32, 16)
    onesi = jnp.ones((16,), jnp.int32)
    zi = jnp.zeros((16,), jnp.int32)
    zf = jnp.zeros((16,), jnp.float32)
    for l in range(16):
        for gb in range(8):
            cnt[l, pl.ds(gb * 16, 16)] = zi
            hist[l, pl.ds(gb * 16, 16)] = zf

    def bchunk(i, carry):
        pltpu.sync_copy(batch.at[pl.ds(i * _BC, _BC)], bch)

        def v(j, carry2):
            bv = bch[pl.ds(j * 16, 16)]
            plsc.addupdate_scatter(cnt, [lane, bv], onesi)
            return carry2

        lax.fori_loop(0, _BC // 16, v, 0)
        return carry

    lax.fori_loop(0, _NBCH, bchunk, 0)

    carry = jnp.int32(0)
    for gb in range(8):
        acc = zi
        for l in range(16):
            acc = acc + cnt[l, pl.ds(gb * 16, 16)]
        blk = plsc.cumsum(acc) + carry
        bnd[pl.ds(gb * 16, 16)] = blk
        carry = blk[15]

    srcs = (a_in, b_in, c_in)

    def issue_in(s, st):
        for r in range(4):
            pltpu.async_copy(epi.at[pl.ds(r * _NP + st, _C)],
                             idx[s * 4 + r], semi[s])
        for k in range(3):
            pltpu.async_copy(srcs[k].at[pl.ds(st, _C)], abc[s * 3 + k],
                             semi[s])

    def wait_in(s):
        for r in range(4):
            pltpu.make_async_copy(epi.at[pl.ds(0, _C)],
                                  idx[s * 4 + r], semi[s]).wait()
        for k in range(3):
            pltpu.make_async_copy(srcs[k].at[pl.ds(0, _C)], abc[s * 3 + k],
                                  semi[s]).wait()

    def compute(s):
        def body(j, carry2):
            o = j * (16 * _U)
            for u in range(_U):
                oo = o + u * 16
                s1 = idx[s * 4 + 0][pl.ds(oo, 16)]
                s2 = idx[s * 4 + 1][pl.ds(oo, 16)]
                e1 = idx[s * 4 + 2][pl.ds(oo, 16)]
                e2 = idx[s * 4 + 3][pl.ds(oo, 16)]
                ys1 = plsc.load_gather(ytab, [s1])
                ys2 = plsc.load_gather(ytab, [s2])
                ye1 = plsc.load_gather(ytab, [e1])
                ye2 = plsc.load_gather(ytab, [e2])
                ap = ye2 - ys2
                bp = ys1 - ys2
                cp = ye1 - ys2
                av = abc[s * 3 + 0][pl.ds(oo, 16)]
                bv = abc[s * 3 + 1][pl.ds(oo, 16)]
                cv = abc[s * 3 + 2][pl.ds(oo, 16)]
                d1 = av * bp - ap * bv
                d2 = av * cp - ap * cv
                d3 = bv * cp - bp * cv
                d4 = d1 - d2 + d3
                cross = (d1 * d2 < -_EPS) & (d3 * d4 < -_EPS)
                xing = jnp.where(cross, 1.0, 0.0).astype(jnp.float32)
                lo = zi
                for stp in (64, 32, 16, 8, 4, 2, 1):
                    mid = lo + stp
                    bb = plsc.load_gather(bnd, [mid - 1])
                    lo = jnp.where(bb <= s1, mid, lo)
                plsc.addupdate_scatter(hist, [lane, lo], xing)
            return carry2

        lax.fori_loop(0, _C // (16 * _U), body, 0)

    issue_in(0, base)
    issue_in(1, base + _C)

    def h_iter(h, carry3):
        st0 = base + (2 * h) * _C
        wait_in(0)
        compute(0)
        issue_in(0, st0 + 2 * _C)
        wait_in(1)
        compute(1)
        issue_in(1, st0 + 3 * _C)
        return carry3

    lax.fori_loop(0, _NCH // 2 - 1, h_iter, 0)
    wait_in(0)
    compute(0)
    wait_in(1)
    compute(1)

    for gb in range(8):
        accf = zf
        for l in range(16):
            accf = accf + hist[l, pl.ds(gb * 16, 16)]
        orow[pl.ds(gb * 16, 16)] = accf
    pltpu.sync_copy(orow, out.at[pl.ds(wid * _NG, _NG)])


def kernel(node_pos, edge_index, apsp, batch_index, edge_pair_index):
    del edge_index, apsp
    xcol = node_pos[:, 0]
    ycol = node_pos[:, 1]
    epi = edge_pair_index.reshape(4 * _NP)
    a, b, c = _pass1(epi, xcol)
    parts = _pass2(epi, ycol, batch_index, a, b, c)
    return parts.reshape(_NW, _NG).sum(axis=0)


# speedup vs baseline: 188.7308x; 1.6084x over previous
import functools

import jax
import jax.numpy as jnp
from jax import lax
from jax.experimental import pallas as pl
from jax.experimental.pallas import tpu as pltpu
from jax.experimental.pallas import tpu_sc as plsc

_EPS = 1e-7
_NN = 100000
_NP = 6400000
_NG = 128
_NC = 2
_NS = 16
_NW = _NC * _NS
_P = _NP // _NW
_C = 400
_NCH = _P // _C
_U = 5
_BC = 2000
_NBCH = _NN // _BC

_mesh = plsc.VectorSubcoreMesh(core_axis_name="c", subcore_axis_name="s")
_cparams = pltpu.CompilerParams(needs_layout_passes=False)


def _worker_id():
    return lax.axis_index("s") * _NC + lax.axis_index("c")


@functools.partial(
    pl.kernel,
    out_type=(
        jax.ShapeDtypeStruct((_NP,), jnp.float32),
        jax.ShapeDtypeStruct((_NP,), jnp.float32),
        jax.ShapeDtypeStruct((_NP,), jnp.float32),
    ),
    mesh=_mesh,
    compiler_params=_cparams,
    scratch_types=[
        pltpu.VMEM((_NN,), jnp.float32),
        [pltpu.VMEM((_C,), jnp.int32)] * 8,
        [pltpu.VMEM((_C,), jnp.float32)] * 6,
        [pltpu.SemaphoreType.DMA] * 4,
    ],
)
def _pass1(epi, xcol, a_out, b_out, c_out, xtab, idx, abc, sems):
    base = _worker_id() * _P
    pltpu.sync_copy(xcol, xtab)
    semi = sems[:2]
    semo = sems[2:]
    outs = (a_out, b_out, c_out)

    def issue_in(s, st):
        for r in range(4):
            pltpu.async_copy(epi.at[pl.ds(r * _NP + st, _C)],
                             idx[s * 4 + r], semi[s])

    def wait_in(s):
        for r in range(4):
            pltpu.make_async_copy(epi.at[pl.ds(0, _C)],
                                  idx[s * 4 + r], semi[s]).wait()

    def issue_out(s, st):
        for k in range(3):
            pltpu.async_copy(abc[s * 3 + k], outs[k].at[pl.ds(st, _C)],
                             semo[s])

    def wait_out(s):
        for k in range(3):
            pltpu.make_async_copy(abc[s * 3 + k], outs[k].at[pl.ds(0, _C)],
                                  semo[s]).wait()

    def compute(s):
        @plsc.parallel_loop(0, _C // 16, unroll=_U)
        def body(j):
            oo = j * 16
            s1 = idx[s * 4 + 0][pl.ds(oo, 16)]
            s2 = idx[s * 4 + 1][pl.ds(oo, 16)]
            e1 = idx[s * 4 + 2][pl.ds(oo, 16)]
            e2 = idx[s * 4 + 3][pl.ds(oo, 16)]
            xs1 = plsc.load_gather(xtab, [s1])
            xs2 = plsc.load_gather(xtab, [s2])
            xe1 = plsc.load_gather(xtab, [e1])
            xe2 = plsc.load_gather(xtab, [e2])
            abc[s * 3 + 0][pl.ds(oo, 16)] = xe2 - xs2
            abc[s * 3 + 1][pl.ds(oo, 16)] = xs1 - xs2
            abc[s * 3 + 2][pl.ds(oo, 16)] = xe1 - xs2

    issue_in(0, base)
    issue_in(1, base + _C)

    def h_iter(h, carry):
        st0 = base + (2 * h) * _C
        wait_in(0)

        @pl.when(h > 0)
        def _wo0():
            wait_out(0)

        compute(0)
        issue_out(0, st0)
        issue_in(0, st0 + 2 * _C)
        st1 = st0 + _C
        wait_in(1)

        @pl.when(h > 0)
        def _wo1():
            wait_out(1)

        compute(1)
        issue_out(1, st1)
        issue_in(1, st1 + 2 * _C)
        return carry

    lax.fori_loop(0, _NCH // 2 - 1, h_iter, 0)

    st0 = base + (_NCH - 2) * _C
    wait_in(0)
    wait_out(0)
    compute(0)
    issue_out(0, st0)
    wait_in(1)
    wait_out(1)
    compute(1)
    issue_out(1, st0 + _C)
    wait_out(0)
    wait_out(1)


@functools.partial(
    pl.kernel,
    out_type=jax.ShapeDtypeStruct((_NW * _NG,), jnp.float32),
    mesh=_mesh,
    compiler_params=_cparams,
    scratch_types=[
        pltpu.VMEM((_NN,), jnp.float32),
        pltpu.VMEM((_BC,), jnp.int32),
        pltpu.VMEM((16, _NG), jnp.int32),
        pltpu.VMEM((_NG,), jnp.int32),
        [pltpu.VMEM((_C,), jnp.int32)] * 8,
        [pltpu.VMEM((_C,), jnp.float32)] * 6,
        pltpu.VMEM((16, _NG), jnp.float32),
        pltpu.VMEM((_NG,), jnp.float32),
        [pltpu.SemaphoreType.DMA] * 2,
    ],
)
def _pass2(epi, ycol, batch, a_in, b_in, c_in, out,
           ytab, bch, cnt, bnd, idx, abc, hist, orow, semi):
    wid = _worker_id()
    base = wid * _P
    pltpu.sync_copy(ycol, ytab)
    lane = lax.iota(jnp.int32, 16)
    onesi = jnp.ones((16,), jnp.int32)
    zi = jnp.zeros((16,), jnp.int32)
    zf = jnp.zeros((16,), jnp.float32)
    for l in range(16):
        for gb in range(8):
            cnt[l, pl.ds(gb * 16, 16)] = zi
            hist[l, pl.ds(gb * 16, 16)] = zf

    def bchunk(i, carry):
        pltpu.sync_copy(batch.at[pl.ds(i * _BC, _BC)], bch)

        @plsc.parallel_loop(0, _BC // 16, unroll=4)
        def v(j):
            bv = bch[pl.ds(j * 16, 16)]
            plsc.addupdate_scatter(cnt, [lane, bv], onesi)
        return carry

    lax.fori_loop(0, _NBCH, bchunk, 0)

    carry = jnp.int32(0)
    for gb in range(8):
        acc = zi
        for l in range(16):
            acc = acc + cnt[l, pl.ds(gb * 16, 16)]
        blk = plsc.cumsum(acc) + carry
        bnd[pl.ds(gb * 16, 16)] = blk
        carry = blk[15]

    srcs = (a_in, b_in, c_in)

    def issue_in(s, st):
        for r in range(4):
            pltpu.async_copy(epi.at[pl.ds(r * _NP + st, _C)],
                             idx[s * 4 + r], semi[s])
        for k in range(3):
            pltpu.async_copy(srcs[k].at[pl.ds(st, _C)], abc[s * 3 + k],
                             semi[s])

    def wait_in(s):
        for r in range(4):
            pltpu.make_async_copy(epi.at[pl.ds(0, _C)],
                                  idx[s * 4 + r], semi[s]).wait()
        for k in range(3):
            pltpu.make_async_copy(srcs[k].at[pl.ds(0, _C)], abc[s * 3 + k],
                                  semi[s]).wait()

    def compute(s):
        @plsc.parallel_loop(0, _C // 16, unroll=_U)
        def body(j):
            oo = j * 16
            s1 = idx[s * 4 + 0][pl.ds(oo, 16)]
            s2 = idx[s * 4 + 1][pl.ds(oo, 16)]
            e1 = idx[s * 4 + 2][pl.ds(oo, 16)]
            e2 = idx[s * 4 + 3][pl.ds(oo, 16)]
            ys1 = plsc.load_gather(ytab, [s1])
            ys2 = plsc.load_gather(ytab, [s2])
            ye1 = plsc.load_gather(ytab, [e1])
            ye2 = plsc.load_gather(ytab, [e2])
            ap = ye2 - ys2
            bp = ys1 - ys2
            cp = ye1 - ys2
            av = abc[s * 3 + 0][pl.ds(oo, 16)]
            bv = abc[s * 3 + 1][pl.ds(oo, 16)]
            cv = abc[s * 3 + 2][pl.ds(oo, 16)]
            d1 = av * bp - ap * bv
            d2 = av * cp - ap * cv
            d3 = bv * cp - bp * cv
            d4 = d1 - d2 + d3
            cross = (d1 * d2 < -_EPS) & (d3 * d4 < -_EPS)
            xing = jnp.where(cross, 1.0, 0.0).astype(jnp.float32)
            lo = zi
            for stp in (64, 32, 16, 8, 4, 2, 1):
                mid = lo + stp
                bb = plsc.load_gather(bnd, [mid - 1])
                lo = jnp.where(bb <= s1, mid, lo)
            plsc.addupdate_scatter(hist, [lane, lo], xing)

    issue_in(0, base)
    issue_in(1, base + _C)

    def h_iter(h, carry3):
        st0 = base + (2 * h) * _C
        wait_in(0)
        compute(0)
        issue_in(0, st0 + 2 * _C)
        wait_in(1)
        compute(1)
        issue_in(1, st0 + 3 * _C)
        return carry3

    lax.fori_loop(0, _NCH // 2 - 1, h_iter, 0)
    wait_in(0)
    compute(0)
    wait_in(1)
    compute(1)

    for gb in range(8):
        accf = zf
        for l in range(16):
            accf = accf + hist[l, pl.ds(gb * 16, 16)]
        orow[pl.ds(gb * 16, 16)] = accf
    pltpu.sync_copy(orow, out.at[pl.ds(wid * _NG, _NG)])


def kernel(node_pos, edge_index, apsp, batch_index, edge_pair_index):
    del edge_index, apsp
    xcol = node_pos[:, 0]
    ycol = node_pos[:, 1]
    epi = edge_pair_index.reshape(4 * _NP)
    a, b, c = _pass1(epi, xcol)
    parts = _pass2(epi, ycol, batch_index, a, b, c)
    return parts.reshape(_NW, _NG).sum(axis=0)


# speedup vs baseline: 189.3277x; 1.0032x over previous
import functools

import jax
import jax.numpy as jnp
from jax import lax
from jax.experimental import pallas as pl
from jax.experimental.pallas import tpu as pltpu
from jax.experimental.pallas import tpu_sc as plsc

_EPS = 1e-7
_NN = 100000
_NP = 6400000
_NG = 128
_NC = 2
_NS = 16
_NW = _NC * _NS
_P = _NP // _NW
_C = 400
_NCH = _P // _C
_U = 5
_BC = 2000
_NBCH = _NN // _BC

_mesh = plsc.VectorSubcoreMesh(core_axis_name="c", subcore_axis_name="s")
_cparams = pltpu.CompilerParams(needs_layout_passes=False)


def _worker_id():
    return lax.axis_index("s") * _NC + lax.axis_index("c")


@functools.partial(
    pl.kernel,
    out_type=(
        jax.ShapeDtypeStruct((_NP,), jnp.float32),
        jax.ShapeDtypeStruct((_NP,), jnp.float32),
        jax.ShapeDtypeStruct((_NP,), jnp.float32),
    ),
    mesh=_mesh,
    compiler_params=_cparams,
    scratch_types=[
        pltpu.VMEM((_NN,), jnp.float32),
        [pltpu.VMEM((_C,), jnp.int32)] * 8,
        [pltpu.VMEM((_C,), jnp.float32)] * 6,
        [pltpu.SemaphoreType.DMA] * 4,
    ],
)
def _pass1(epi, xcol, a_out, b_out, c_out, xtab, idx, abc, sems):
    base = _worker_id() * _P
    pltpu.sync_copy(xcol, xtab)
    semi = sems[:2]
    semo = sems[2:]
    outs = (a_out, b_out, c_out)

    def issue_in(s, st):
        for r in range(4):
            pltpu.async_copy(epi.at[pl.ds(r * _NP + st, _C)],
                             idx[s * 4 + r], semi[s])

    def wait_in(s):
        for r in range(4):
            pltpu.make_async_copy(epi.at[pl.ds(0, _C)],
                                  idx[s * 4 + r], semi[s]).wait()

    def issue_out(s, st):
        for k in range(3):
            pltpu.async_copy(abc[s * 3 + k], outs[k].at[pl.ds(st, _C)],
                             semo[s])

    def wait_out(s):
        for k in range(3):
            pltpu.make_async_copy(abc[s * 3 + k], outs[k].at[pl.ds(0, _C)],
                                  semo[s]).wait()

    def compute(s):
        @plsc.parallel_loop(0, _C // 16, unroll=_U)
        def body(j):
            oo = j * 16
            s1 = idx[s * 4 + 0][pl.ds(oo, 16)]
            s2 = idx[s * 4 + 1][pl.ds(oo, 16)]
            e1 = idx[s * 4 + 2][pl.ds(oo, 16)]
            e2 = idx[s * 4 + 3][pl.ds(oo, 16)]
            xs1 = plsc.load_gather(xtab, [s1])
            xs2 = plsc.load_gather(xtab, [s2])
            xe1 = plsc.load_gather(xtab, [e1])
            xe2 = plsc.load_gather(xtab, [e2])
            abc[s * 3 + 0][pl.ds(oo, 16)] = xe2 - xs2
            abc[s * 3 + 1][pl.ds(oo, 16)] = xs1 - xs2
            abc[s * 3 + 2][pl.ds(oo, 16)] = xe1 - xs2

    issue_in(0, base)
    issue_in(1, base + _C)

    def h_iter(h, carry):
        st0 = base + (2 * h) * _C
        wait_in(0)

        @pl.when(h > 0)
        def _wo0():
            wait_out(0)

        compute(0)
        issue_out(0, st0)
        issue_in(0, st0 + 2 * _C)
        st1 = st0 + _C
        wait_in(1)

        @pl.when(h > 0)
        def _wo1():
            wait_out(1)

        compute(1)
        issue_out(1, st1)
        issue_in(1, st1 + 2 * _C)
        return carry

    lax.fori_loop(0, _NCH // 2 - 1, h_iter, 0)

    st0 = base + (_NCH - 2) * _C
    wait_in(0)
    wait_out(0)
    compute(0)
    issue_out(0, st0)
    wait_in(1)
    wait_out(1)
    compute(1)
    issue_out(1, st0 + _C)
    wait_out(0)
    wait_out(1)


@functools.partial(
    pl.kernel,
    out_type=jax.ShapeDtypeStruct((_NW * _NG,), jnp.float32),
    mesh=_mesh,
    compiler_params=_cparams,
    scratch_types=[
        pltpu.VMEM((_NN,), jnp.float32),
        pltpu.VMEM((_BC,), jnp.int32),
        pltpu.VMEM((16, _NG), jnp.int32),
        pltpu.VMEM((_NG,), jnp.int32),
        [pltpu.VMEM((_C,), jnp.int32)] * 8,
        [pltpu.VMEM((_C,), jnp.float32)] * 6,
        pltpu.VMEM((16, _NG), jnp.float32),
        pltpu.VMEM((_NG,), jnp.float32),
        [pltpu.SemaphoreType.DMA] * 2,
    ],
)
def _pass2(epi, ycol, batch, a_in, b_in, c_in, out,
           ytab, bch, cnt, bnd, idx, abc, hist, orow, semi):
    wid = _worker_id()
    base = wid * _P
    pltpu.sync_copy(ycol, ytab)
    lane = lax.iota(jnp.int32, 16)
    onesi = jnp.ones((16,), jnp.int32)
    zi = jnp.zeros((16,), jnp.int32)
    zf = jnp.zeros((16,), jnp.float32)
    for l in range(16):
        for gb in range(8):
            cnt[l, pl.ds(gb * 16, 16)] = zi
            hist[l, pl.ds(gb * 16, 16)] = zf

    def bchunk(i, carry):
        pltpu.sync_copy(batch.at[pl.ds(i * _BC, _BC)], bch)

        @plsc.parallel_loop(0, _BC // 16, unroll=4)
        def v(j):
            bv = bch[pl.ds(j * 16, 16)]
            plsc.addupdate_scatter(cnt, [lane, bv], onesi)
        return carry

    lax.fori_loop(0, _NBCH, bchunk, 0)

    carry = jnp.int32(0)
    for gb in range(8):
        acc = zi
        for l in range(16):
            acc = acc + cnt[l, pl.ds(gb * 16, 16)]
        blk = plsc.cumsum(acc) + carry
        bnd[pl.ds(gb * 16, 16)] = blk
        carry = blk[15]

    srcs = (a_in, b_in, c_in)

    def issue_in(s, st):
        for r in range(4):
            pltpu.async_copy(epi.at[pl.ds(r * _NP + st, _C)],
                             idx[s * 4 + r], semi[s])
        for k in range(3):
            pltpu.async_copy(srcs[k].at[pl.ds(st, _C)], abc[s * 3 + k],
                             semi[s])

    def wait_in(s):
        for r in range(4):
            pltpu.make_async_copy(epi.at[pl.ds(0, _C)],
                                  idx[s * 4 + r], semi[s]).wait()
        for k in range(3):
            pltpu.make_async_copy(srcs[k].at[pl.ds(0, _C)], abc[s * 3 + k],
                                  semi[s]).wait()

    def compute(s):
        @plsc.parallel_loop(0, _C // 16, unroll=_U)
        def body(j):
            oo = j * 16
            s1 = idx[s * 4 + 0][pl.ds(oo, 16)]
            s2 = idx[s * 4 + 1][pl.ds(oo, 16)]
            e1 = idx[s * 4 + 2][pl.ds(oo, 16)]
            e2 = idx[s * 4 + 3][pl.ds(oo, 16)]
            ys1 = plsc.load_gather(ytab, [s1])
            ys2 = plsc.load_gather(ytab, [s2])
            ye1 = plsc.load_gather(ytab, [e1])
            ye2 = plsc.load_gather(ytab, [e2])
            ap = ye2 - ys2
            bp = ys1 - ys2
            cp = ye1 - ys2
            av = abc[s * 3 + 0][pl.ds(oo, 16)]
            bv = abc[s * 3 + 1][pl.ds(oo, 16)]
            cv = abc[s * 3 + 2][pl.ds(oo, 16)]
            d1 = av * bp - ap * bv
            d2 = av * cp - ap * cv
            d3 = bv * cp - bp * cv
            d4 = d1 - d2 + d3
            cross = (d1 * d2 < -_EPS) & (d3 * d4 < -_EPS)
            xing = jnp.where(cross, 1.0, 0.0).astype(jnp.float32)
            lo = zi
            for stp in (64, 32, 16, 8, 4, 2, 1):
                mid = lo + stp
                bb = plsc.load_gather(bnd, [mid - 1])
                lo = jnp.where(bb <= s1, mid, lo)
            plsc.addupdate_scatter(hist, [lane, lo], xing)

    issue_in(0, base)
    issue_in(1, base + _C)

    def h_iter(h, carry3):
        st0 = base + (2 * h) * _C
        wait_in(0)
        compute(0)
        issue_in(0, st0 + 2 * _C)
        wait_in(1)
        compute(1)
        issue_in(1, st0 + 3 * _C)
        return carry3

    lax.fori_loop(0, _NCH // 2 - 1, h_iter, 0)
    wait_in(0)
    compute(0)
    wait_in(1)
    compute(1)

    for gb in range(8):
        accf = zf
        for l in range(16):
            accf = accf + hist[l, pl.ds(gb * 16, 16)]
        orow[pl.ds(gb * 16, 16)] = accf
    pltpu.sync_copy(orow, out.at[pl.ds(wid * _NG, _NG)])


def kernel(node_pos, edge_index, apsp, batch_index, edge_pair_index):
    del edge_index, apsp
    xcol = node_pos @ jnp.array([1.0, 0.0], dtype=jnp.float32)
    ycol = node_pos @ jnp.array([0.0, 1.0], dtype=jnp.float32)
    epi = edge_pair_index.reshape(4 * _NP)
    a, b, c = _pass1(epi, xcol)
    parts = _pass2(epi, ycol, batch_index, a, b, c)
    return parts.reshape(_NW, _NG).sum(axis=0)
